# Initial kernel scaffold; baseline (speedup 1.0000x reference)
import jax
import jax.numpy as jnp
from jax.experimental import pallas as pl


def kernel(features_1, edge_index_1, W1, b1, W2, b2, W3, b3, Watt, Wfc, bfc, Wsc, bsc):
    raise NotImplementedError("write your pallas kernel here")


# speedup vs baseline: 5.4319x; 5.4319x over previous
import functools

import jax
import jax.numpy as jnp
from jax import lax
from jax.experimental import pallas as pl
from jax.experimental.pallas import tpu as pltpu
from jax.experimental.pallas import tpu_sc as plsc

N, E = 10000, 320000
F0, F1, F2, F3, BN = 128, 128, 64, 32, 16
ROWS = 1000

NC, NS = 2, 16
NW = NC * NS
CH = 128
NCHUNKS = 80
EPW = NCHUNKS * CH
E_PAD = NW * EPW
NROWS = 10240
RPT = NROWS // NS

_mesh = plsc.VectorSubcoreMesh(core_axis_name="c", subcore_axis_name="s",
                               num_cores=NC, num_subcores=NS)


def _sc_spmm(g, src_p, dst_p, zrows):
    F = g.shape[1]

    @functools.partial(
        pl.kernel,
        out_type=jax.ShapeDtypeStruct((NC, NROWS, F), jnp.float32),
        mesh=_mesh,
        scratch_types=[
            pltpu.VMEM_SHARED((NROWS, F), jnp.float32),
            pltpu.VMEM((CH,), jnp.int32),
            pltpu.VMEM((CH,), jnp.int32),
            pltpu.VMEM((CH, F), jnp.float32),
            pltpu.SemaphoreType.DMA,
        ],
    )
    def k(g_h, src_h, dst_h, z_h, out_h, acc_sh, sidx, didx, rows, sem):
        c = lax.axis_index("c")
        s = lax.axis_index("s")
        base_r = s * RPT
        for j in range(RPT // 64):
            pltpu.sync_copy(z_h, acc_sh.at[pl.ds(base_r + j * 64, 64)])
        plsc.subcore_barrier()
        base_e = (s * NC + c) * EPW

        def step(i, carry):
            off = base_e + i * CH
            pltpu.sync_copy(src_h.at[pl.ds(off, CH)], sidx)
            pltpu.sync_copy(dst_h.at[pl.ds(off, CH)], didx)
            pltpu.async_copy(g_h.at[sidx], rows, sem).wait()
            pltpu.sync_copy(rows, acc_sh.at[didx], add=True)
            return carry

        lax.fori_loop(0, NCHUNKS, step, 0)
        plsc.subcore_barrier()
        pltpu.sync_copy(acc_sh.at[pl.ds(base_r, RPT)],
                        out_h.at[c, pl.ds(base_r, RPT)])

    return k(g, src_p, dst_p, zrows)


def _sc_deg(dst_p, ones, zvec):
    @functools.partial(
        pl.kernel,
        out_type=jax.ShapeDtypeStruct((NC, NROWS), jnp.float32),
        mesh=_mesh,
        scratch_types=[
            pltpu.VMEM_SHARED((NROWS,), jnp.float32),
            pltpu.VMEM((CH,), jnp.int32),
            pltpu.VMEM((CH,), jnp.float32),
        ],
    )
    def k(dst_h, ones_h, z_h, out_h, cnt_sh, didx, ones_v):
        c = lax.axis_index("c")
        s = lax.axis_index("s")
        base_r = s * RPT
        pltpu.sync_copy(z_h, cnt_sh.at[pl.ds(base_r, RPT)])
        pltpu.sync_copy(ones_h, ones_v)
        plsc.subcore_barrier()
        base_e = (s * NC + c) * EPW

        def step(i, carry):
            off = base_e + i * CH
            pltpu.sync_copy(dst_h.at[pl.ds(off, CH)], didx)
            pltpu.sync_copy(ones_v, cnt_sh.at[didx], add=True)
            return carry

        lax.fori_loop(0, NCHUNKS, step, 0)
        plsc.subcore_barrier()
        pltpu.sync_copy(cnt_sh.at[pl.ds(base_r, RPT)],
                        out_h.at[c, pl.ds(base_r, RPT)])

    return k(dst_p, ones, zvec)


def _dis_from(cnt_ref):
    c = cnt_ref[0, 0, 0, :] + cnt_ref[1, 0, 0, :]
    return lax.rsqrt(c + 1.0)[:, None]


def _mm1_body(x_ref, w_ref, cnt_ref, g_ref):
    h = jnp.dot(x_ref[...], w_ref[...], preferred_element_type=jnp.float32)
    g_ref[...] = h * _dis_from(cnt_ref)


def _mm1(x, W, cnt4):
    return pl.pallas_call(
        _mm1_body,
        grid=(N // ROWS,),
        in_specs=[
            pl.BlockSpec((ROWS, x.shape[1]), lambda i: (i, 0)),
            pl.BlockSpec(W.shape, lambda i: (0, 0)),
            pl.BlockSpec((2, 1, 1, ROWS), lambda i: (0, i, 0, 0)),
        ],
        out_specs=pl.BlockSpec((ROWS, W.shape[1]), lambda i: (i, 0)),
        out_shape=jax.ShapeDtypeStruct((N, W.shape[1]), jnp.float32),
    )(x, W, cnt4)


def _layer_body(g_ref, acc_ref, cnt_ref, b_ref, w_ref, out_ref):
    dis = _dis_from(cnt_ref)
    f = dis * (acc_ref[0] + acc_ref[1] + g_ref[...]) + b_ref[...][None, :]
    f = jnp.maximum(f, 0.0)
    h = jnp.dot(f, w_ref[...], preferred_element_type=jnp.float32)
    out_ref[...] = h * dis


def _layer(g, acc, cnt4, b, W):
    K = g.shape[1]
    return pl.pallas_call(
        _layer_body,
        grid=(N // ROWS,),
        in_specs=[
            pl.BlockSpec((ROWS, K), lambda i: (i, 0)),
            pl.BlockSpec((2, ROWS, K), lambda i: (0, i, 0)),
            pl.BlockSpec((2, 1, 1, ROWS), lambda i: (0, i, 0, 0)),
            pl.BlockSpec(b.shape, lambda i: (0,)),
            pl.BlockSpec(W.shape, lambda i: (0, 0)),
        ],
        out_specs=pl.BlockSpec((ROWS, W.shape[1]), lambda i: (i, 0)),
        out_shape=jax.ShapeDtypeStruct((N, W.shape[1]), jnp.float32),
    )(g, acc, cnt4, b, W)


def _final_body(g_ref, acc_ref, cnt_ref, b_ref, watt_ref, wfc_ref, bfc_ref,
                wsc_ref, bsc_ref, out_ref):
    dis = lax.rsqrt(cnt_ref[0] + cnt_ref[1] + 1.0)[:, None]
    f = dis * (acc_ref[0][:, :F3] + acc_ref[1][:, :F3] + g_ref[...][:, :F3])         + b_ref[...][None, :]
    e1 = jnp.dot(f, watt_ref[...], preferred_element_type=jnp.float32)
    gc = jnp.mean(e1, axis=0, keepdims=True)
    tg = jnp.tanh(gc)
    sv = jnp.dot(f, tg.reshape(F3, 1), preferred_element_type=jnp.float32)
    sig = jax.nn.sigmoid(sv)
    rep = lax.dot_general(f, sig, (((0,), (0,)), ((), ())),
                          preferred_element_type=jnp.float32)
    sc = jnp.dot(rep.reshape(1, F3), wfc_ref[...], preferred_element_type=jnp.float32)
    sc = jnp.maximum(sc + bfc_ref[...][None, :], 0.0)
    s = jnp.sum(sc.reshape(BN, 1) * wsc_ref[...], axis=0, keepdims=True)
    out_ref[...] = jax.nn.sigmoid(s + bsc_ref[...][None, :])


def _final(g, acc, cnt2, b, Watt, Wfc, bfc, Wsc, bsc):
    return pl.pallas_call(
        _final_body,
        grid=(1,),
        in_specs=[
            pl.BlockSpec((N, F0), lambda i: (0, 0)),
            pl.BlockSpec((2, N, F0), lambda i: (0, 0, 0)),
            pl.BlockSpec((2, N), lambda i: (0, 0)),
            pl.BlockSpec((F3,), lambda i: (0,)),
            pl.BlockSpec((F3, F3), lambda i: (0, 0)),
            pl.BlockSpec((F3, BN), lambda i: (0, 0)),
            pl.BlockSpec((BN,), lambda i: (0,)),
            pl.BlockSpec((BN, 1), lambda i: (0, 0)),
            pl.BlockSpec((1,), lambda i: (0,)),
        ],
        out_specs=pl.BlockSpec((1, 1), lambda i: (0, 0)),
        out_shape=jax.ShapeDtypeStruct((1, 1), jnp.float32),
    )(g, acc, cnt2, b, Watt, Wfc, bfc, Wsc, bsc)


def kernel(features_1, edge_index_1, W1, b1, W2, b2, W3, b3, Watt, Wfc, bfc, Wsc, bsc):
    src = edge_index_1[0]
    dst = edge_index_1[1]
    pad = E_PAD - E
    src_p = jnp.concatenate([src, jnp.zeros((pad,), jnp.int32)])
    dst_p = jnp.concatenate([dst, jnp.full((pad,), N, jnp.int32)])

    ones = jnp.ones((CH,), jnp.float32)
    zvec = jnp.zeros((RPT,), jnp.float32)

    cnt = _sc_deg(dst_p, ones, zvec)
    cnt4 = cnt[:, :N].reshape(2, N // ROWS, 1, ROWS)
    cnt2 = cnt[:, :N]

    W2p = jnp.pad(W2, ((0, 0), (0, F0 - F2)))
    b2p = jnp.pad(b2, (0, F0 - F2))
    W3p = jnp.pad(W3, ((0, F0 - F2), (0, F0 - F3)))
    b3p = jnp.pad(b3, (0, F0 - F3))
    z64 = jnp.zeros((64, F0), jnp.float32)

    g1 = _mm1(features_1, W1, cnt4)
    acc1 = _sc_spmm(g1, src_p, dst_p, z64)
    g2 = _layer(g1, acc1, cnt4, b1, W2p)
    acc2 = _sc_spmm(g2, src_p, dst_p, z64)
    g3 = _layer(g2, acc2, cnt4, b2p, W3p)
    acc3 = _sc_spmm(g3, src_p, dst_p, z64)
    return _final(g3, acc3, cnt2, b3, Watt, Wfc, bfc, Wsc, bsc)
